# baseline (speedup 1.0000x reference)
import functools

import jax
import jax.numpy as jnp
from jax.experimental import pallas as pl
from jax.experimental.pallas import tpu as pltpu


R = 1024
K = 3


def _pos_add_body(x_hbm, pos_hbm, out_hbm, posbuf, xbuf, obuf,
                  pos_sems, rd_sems, wr_sems, *, n_chunks, pos_chunks):
    def pos_copy(c):
        return pltpu.make_async_copy(
            pos_hbm.at[pl.ds(c * R, R)], posbuf.at[pl.ds(c * R, R)],
            pos_sems.at[c])

    def rd_copy(i, slot):
        return pltpu.make_async_copy(
            x_hbm.at[pl.ds(i * R, R)], xbuf.at[slot], rd_sems.at[slot])

    def wr_copy(i, slot):
        return pltpu.make_async_copy(
            obuf.at[slot], out_hbm.at[pl.ds(i * R, R)], wr_sems.at[slot])

    for c in range(pos_chunks):
        pos_copy(c).start()
    for i in range(K):
        rd_copy(i, i).start()

    def step(i, _):
        slot = jax.lax.rem(i, K)
        pc = jax.lax.rem(i, pos_chunks)

        @pl.when(i < pos_chunks)
        def _():
            pos_copy(pc).wait()

        rd_copy(i, slot).wait()

        @pl.when(i >= K)
        def _():
            wr_copy(i - K, slot).wait()

        obuf[slot] = xbuf[slot] + posbuf[pl.ds(pc * R, R), :]
        wr_copy(i, slot).start()

        @pl.when(i + K < n_chunks)
        def _():
            rd_copy(i + K, slot).start()

        return 0

    jax.lax.fori_loop(0, n_chunks, step, 0)

    for j in range(K):
        i = n_chunks - K + j
        wr_copy(i, i % K).wait()


def kernel(x, pos_table):
    batch, seq_len, d_model = x.shape
    rows = batch * seq_len
    n_chunks = rows // R
    pos_chunks = seq_len // R
    xf = x.reshape(rows, d_model)
    pos = pos_table[:seq_len]

    body = functools.partial(_pos_add_body, n_chunks=n_chunks,
                             pos_chunks=pos_chunks)
    out = pl.pallas_call(
        body,
        in_specs=[
            pl.BlockSpec(memory_space=pl.ANY),
            pl.BlockSpec(memory_space=pl.ANY),
        ],
        out_specs=pl.BlockSpec(memory_space=pl.ANY),
        out_shape=jax.ShapeDtypeStruct((rows, d_model), x.dtype),
        scratch_shapes=[
            pltpu.VMEM((seq_len, d_model), x.dtype),
            pltpu.VMEM((K, R, d_model), x.dtype),
            pltpu.VMEM((K, R, d_model), x.dtype),
            pltpu.SemaphoreType.DMA((pos_chunks,)),
            pltpu.SemaphoreType.DMA((K,)),
            pltpu.SemaphoreType.DMA((K,)),
        ],
    )(xf, pos)
    return out.reshape(batch, seq_len, d_model)
